# baseline (speedup 1.0000x reference)
import jax
import jax.numpy as jnp
from jax import lax
from jax.experimental import pallas as pl
from jax.experimental.pallas import tpu as pltpu
from jax.experimental.pallas import tpu_sc as plsc

B = 1024
SEQ = 77
D = 512
N_PRE = 7
N_SHARE = 16
N_ATTR = 15
N_SUF = 39
N_HEAD = N_PRE + N_SHARE
SUF_START = N_HEAD + N_ATTR
NC = 2
NS = 16
NW = NC * NS
BPW = B // NW
NBUF = 3


def _sc_body(prefix_hbm, suffix_hbm, share_hbm, attr_hbm, vid_hbm, out_hbm,
             head_v, suffix_v, attr_v, vid_v,
             sem_t, sem_a0, sem_a1, sem_a2, sem_b0, sem_b1, sem_b2):
    sem_a = [sem_a0, sem_a1, sem_a2]
    sem_b = [sem_b0, sem_b1, sem_b2]
    wid = lax.axis_index("s") * NC + lax.axis_index("c")
    base = wid * BPW
    pltpu.sync_copy(vid_hbm.at[pl.ds(base, BPW)], vid_v)
    pltpu.sync_copy(suffix_hbm, suffix_v)
    for v in range(3):
        pltpu.sync_copy(prefix_hbm.at[v, 0],
                        head_v.at[v, pl.ds(0, N_PRE)])
        pltpu.sync_copy(share_hbm,
                        head_v.at[v, pl.ds(N_PRE, N_SHARE)])

    def attr_in(i):
        return pltpu.async_copy(
            attr_hbm.at[base + i], attr_v.at[i % NBUF], sem_a[i % NBUF])

    h_in = [attr_in(0), attr_in(1), None]
    h_out = [None, None, None]
    h_tab = []
    for i in range(BPW):
        b = base + i
        p = i % NBUF
        v = vid_v[pl.ds((i // 16) * 16, 16)][i % 16]
        h_tab.append(pltpu.async_copy(
            head_v.at[v], out_hbm.at[b, pl.ds(0, N_HEAD)], sem_t))
        h_tab.append(pltpu.async_copy(
            suffix_v.at[v, 0], out_hbm.at[b, pl.ds(SUF_START, N_SUF)],
            sem_t))
        h_in[p].wait()
        h_out[p] = pltpu.async_copy(
            attr_v.at[p], out_hbm.at[b, pl.ds(N_HEAD, N_ATTR)], sem_b[p])
        if i + 2 < BPW:
            q = (i - 1) % NBUF
            if h_out[q] is not None:
                h_out[q].wait()
                h_out[q] = None
            h_in[q] = attr_in(i + 2)
    for h in h_out:
        if h is not None:
            h.wait()
    for h in h_tab:
        h.wait()


@jax.jit
def _sc_call(prefix, suffix, share, attr, vid):
    mesh = plsc.VectorSubcoreMesh(core_axis_name="c", subcore_axis_name="s")
    f = pl.kernel(
        _sc_body,
        out_type=jax.ShapeDtypeStruct((B, SEQ, D), jnp.float32),
        mesh=mesh,
        scratch_types=[
            pltpu.VMEM((3, N_HEAD, D), jnp.float32),
            pltpu.VMEM((3, 1, N_SUF, D), jnp.float32),
            pltpu.VMEM((NBUF, N_ATTR, D), jnp.float32),
            pltpu.VMEM((BPW,), jnp.int32),
            pltpu.SemaphoreType.DMA,
            pltpu.SemaphoreType.DMA,
            pltpu.SemaphoreType.DMA,
            pltpu.SemaphoreType.DMA,
            pltpu.SemaphoreType.DMA,
            pltpu.SemaphoreType.DMA,
            pltpu.SemaphoreType.DMA,
        ],
        compiler_params=pltpu.CompilerParams(use_tc_tiling_on_sc=False),
    )
    return f(prefix, suffix, share, attr, vid)


def kernel(attribute, viewids, token_prefix, token_suffix, share_vectors):
    return _sc_call(token_prefix, token_suffix, share_vectors, attribute,
                    viewids.astype(jnp.int32))


# speedup vs baseline: 1.8681x; 1.8681x over previous
import jax
import jax.numpy as jnp
from jax import lax
from jax.experimental import pallas as pl
from jax.experimental.pallas import tpu as pltpu
from jax.experimental.pallas import tpu_sc as plsc

B = 1024
SEQ = 77
D = 512
N_PRE = 7
N_SHARE = 16
N_ATTR = 15
N_SUF = 39
N_HEAD = 16
MID_OFF = 16
MID = 24
W3_OFF = 40
W3 = 32
TAIL_OFF = 72
N_TAIL = 5
NC = 2
NS = 16
NW = NC * NS
BPW = B // NW
GI = 64
NSTEPS = B // GI


def _sc_body(head_hbm, suf32_hbm, tail_hbm, vid_hbm, out_hbm,
             head_v, suf32_v, tail_v, vid_v, sem_t):
    wid = lax.axis_index("s") * NC + lax.axis_index("c")
    base = wid * BPW
    pltpu.sync_copy(vid_hbm, vid_v)
    pltpu.sync_copy(head_hbm, head_v)
    pltpu.sync_copy(suf32_hbm, suf32_v)
    pltpu.sync_copy(tail_hbm, tail_v)
    hs = []
    for i in range(BPW):
        b = base + i
        v = vid_v[pl.ds(base + (i // 16) * 16, 16)][i % 16]
        hs.append(pltpu.async_copy(
            head_v.at[v], out_hbm.at[b, pl.ds(0, N_HEAD)], sem_t))
        hs.append(pltpu.async_copy(
            suf32_v.at[v], out_hbm.at[b, pl.ds(W3_OFF, W3)], sem_t))
        hs.append(pltpu.async_copy(
            tail_v.at[v], out_hbm.at[b, pl.ds(TAIL_OFF, N_TAIL)], sem_t))
    for h in hs:
        h.wait()


@jax.jit
def _sc_scatter(head16, suf32, tail5, vid):
    mesh = plsc.VectorSubcoreMesh(core_axis_name="c", subcore_axis_name="s")
    f = pl.kernel(
        _sc_body,
        out_type=jax.ShapeDtypeStruct((B, SEQ, D), jnp.float32),
        mesh=mesh,
        scratch_types=[
            pltpu.VMEM((3, N_HEAD, D), jnp.float32),
            pltpu.VMEM((3, W3, D), jnp.float32),
            pltpu.VMEM((3, N_TAIL, D), jnp.float32),
            pltpu.VMEM((B,), jnp.int32),
            pltpu.SemaphoreType.DMA,
        ],
    )
    return f(head16, suf32, tail5, vid)


def _tc_body(vid_ref, attr_ref, share7_ref, suf2_ref, out1_ref, out_ref,
             buf, sem_o):
    g = pl.program_id(0)
    slot = g % 2

    def dst(step):
        return out_ref.at[pl.ds(step * GI, GI), pl.ds(MID_OFF, MID)]

    @pl.when(g >= 2)
    def _drain_prev():
        pltpu.make_async_copy(buf.at[slot], dst(g), sem_o).wait()

    sh = share7_ref[...]
    buf[slot, :, 0:7, :] = jnp.broadcast_to(sh[None], (GI, 7, D))
    buf[slot, :, 7:7 + N_ATTR, :] = attr_ref[...]
    s0 = suf2_ref[0]
    s1 = suf2_ref[1]
    s2 = suf2_ref[2]
    for i in range(GI):
        v = vid_ref[g * GI + i]
        sel = jnp.where(v == 0, s0, jnp.where(v == 1, s1, s2))
        buf[slot, i, 22:24, :] = sel
    pltpu.make_async_copy(buf.at[slot], dst(g), sem_o).start()

    @pl.when(g == NSTEPS - 1)
    def _drain_last():
        pltpu.make_async_copy(buf.at[1 - slot], dst(g), sem_o).wait()
        pltpu.make_async_copy(buf.at[slot], dst(g), sem_o).wait()


@jax.jit
def _tc_finish(out1, attr, share7, suf2, vid):
    return pl.pallas_call(
        _tc_body,
        grid=(NSTEPS,),
        out_shape=jax.ShapeDtypeStruct((B, SEQ, D), jnp.float32),
        in_specs=[
            pl.BlockSpec((B,), lambda g: (0,), memory_space=pltpu.SMEM),
            pl.BlockSpec((GI, N_ATTR, D), lambda g: (g, 0, 0)),
            pl.BlockSpec((N_PRE, D), lambda g: (0, 0)),
            pl.BlockSpec((3, 2, D), lambda g: (0, 0, 0)),
            pl.BlockSpec(memory_space=pl.ANY),
        ],
        out_specs=pl.BlockSpec(memory_space=pl.ANY),
        scratch_shapes=[
            pltpu.VMEM((2, GI, MID, D), jnp.float32),
            pltpu.SemaphoreType.DMA,
        ],
        input_output_aliases={4: 0},
    )(vid, attr, share7, suf2, out1)


def kernel(attribute, viewids, token_prefix, token_suffix, share_vectors):
    pre3 = token_prefix[:, 0]
    suf3 = token_suffix[:, 0]
    shareb = jnp.broadcast_to(share_vectors[None], (3, N_SHARE, D))
    head16 = jnp.concatenate([pre3, shareb[:, :9]], axis=1)
    suf32 = suf3[:, 2:34]
    tail5 = suf3[:, 34:]
    suf2 = suf3[:, :2]
    share7 = share_vectors[9:]
    vid = viewids.astype(jnp.int32)
    out1 = _sc_scatter(head16, suf32, tail5, vid)
    return _tc_finish(out1, attribute, share7, suf2, vid)
